# baseline (speedup 1.0000x reference)
import functools

import jax
import jax.numpy as jnp
import numpy as np
from jax import lax
from jax.experimental import pallas as pl
from jax.experimental.pallas import tpu as pltpu
from jax.experimental.pallas import tpu_sc as plsc

S, B, C = 200, 1024, 64
TOK = S * B
TSCALE = 0.005
EPS = 1e-5

NC, NS = 2, 16
NW = NC * NS
GRP = 128
GPW = TOK // (NW * GRP)
TPW = TOK // NW


def _half_ln(x, w2, b2):
    lo = x[:, :C]
    hi = x[:, C:]
    mu_lo = jnp.mean(lo, axis=-1, keepdims=True)
    mu_hi = jnp.mean(hi, axis=-1, keepdims=True)
    var_lo = jnp.mean((lo - mu_lo) ** 2, axis=-1, keepdims=True)
    var_hi = jnp.mean((hi - mu_hi) ** 2, axis=-1, keepdims=True)
    y_lo = (lo - mu_lo) * lax.rsqrt(var_lo + EPS)
    y_hi = (hi - mu_hi) * lax.rsqrt(var_hi + EPS)
    return jnp.concatenate([y_lo, y_hi], axis=-1) * w2 + b2


def _ln_rows_body(t_ref, w_ref, b_ref, o_ref):
    o_ref[...] = _half_ln(t_ref[...], w_ref[...], b_ref[...])


def _prenorm(table, w, b, block):
    n = table.shape[0]
    t2 = table.reshape(n // 2, 2 * C)
    w2 = jnp.concatenate([w, w]).reshape(1, 2 * C)
    b2 = jnp.concatenate([b, b]).reshape(1, 2 * C)
    out2 = pl.pallas_call(
        _ln_rows_body,
        grid=(n // 2 // block,),
        in_specs=[
            pl.BlockSpec((block, 2 * C), lambda i: (i, 0)),
            pl.BlockSpec((1, 2 * C), lambda i: (0, 0)),
            pl.BlockSpec((1, 2 * C), lambda i: (0, 0)),
        ],
        out_specs=pl.BlockSpec((block, 2 * C), lambda i: (i, 0)),
        out_shape=jax.ShapeDtypeStruct((n // 2, 2 * C), jnp.float32),
    )(t2, w2, b2)
    return out2.reshape(n, C)


def _make_gather_sum():
    mesh = plsc.VectorSubcoreMesh(core_axis_name="c", subcore_axis_name="s")

    @functools.partial(
        pl.kernel,
        mesh=mesh,
        compiler_params=pltpu.CompilerParams(use_tc_tiling_on_sc=False),
        out_type=jax.ShapeDtypeStruct((TOK, C), jnp.float32),
        scratch_types=[
            pltpu.VMEM((TPW,), jnp.int32),
            pltpu.VMEM((TPW,), jnp.int32),
            pltpu.VMEM((TPW,), jnp.int32),
            pltpu.VMEM((TPW,), jnp.int32),
            pltpu.VMEM((GRP, C), jnp.float32),
            pltpu.VMEM((GRP, C), jnp.float32),
            pltpu.VMEM((GRP, C), jnp.float32),
            pltpu.VMEM((GRP, C), jnp.float32),
            pltpu.SemaphoreType.DMA,
        ],
    )
    def gather_sum(st, ct, nt, tt, ixs, ixc, ixn, ixt, out,
                   vs, vc, vn, vt, rs, rc, rn, rt, sem):
        cid = lax.axis_index("c")
        sid = lax.axis_index("s")
        wid = sid * NC + cid
        tok0 = wid * TPW

        pltpu.sync_copy(ixs.at[pl.ds(tok0, TPW)], vs)
        pltpu.sync_copy(ixc.at[pl.ds(tok0, TPW)], vc)
        pltpu.sync_copy(ixn.at[pl.ds(tok0, TPW)], vn)
        pltpu.sync_copy(ixt.at[pl.ds(tok0, TPW)], vt)

        def group(g, carry):
            off = g * GRP
            tok = tok0 + off
            c1 = pltpu.async_copy(st.at[vs.at[pl.ds(off, GRP)]], rs, sem)
            c2 = pltpu.async_copy(ct.at[vc.at[pl.ds(off, GRP)]], rc, sem)
            c3 = pltpu.async_copy(nt.at[vn.at[pl.ds(off, GRP)]], rn, sem)
            c4 = pltpu.async_copy(tt.at[vt.at[pl.ds(off, GRP)]], rt, sem)
            c1.wait()
            c2.wait()
            c3.wait()
            c4.wait()

            def tok_body(j, cc):
                for q in range(C // 16):
                    sl = pl.ds(q * 16, 16)
                    rs[j, sl] = rs[j, sl] + rc[j, sl] + rn[j, sl] + rt[j, sl]
                return cc

            lax.fori_loop(0, GRP, tok_body, 0)
            pltpu.sync_copy(rs, out.at[pl.ds(tok, GRP)])
            return carry

        lax.fori_loop(0, GPW, group, 0)

    return gather_sum


_gather_sum = _make_gather_sum()


def _pose_body(p_ref, part_ref, W_ref, pb_ref, w_ref, b_ref, sc_ref, o_ref):
    x = p_ref[...] * sc_ref[...]
    pe = jnp.dot(x, W_ref[...], preferred_element_type=jnp.float32) + pb_ref[...]
    o_ref[...] = _half_ln(pe, w_ref[...], b_ref[...]) + part_ref[...]


_POSE_R = 2048


def _pose_add(pose2, partial2, W2, pb2, pw2, pbb2, scale2):
    n2 = TOK // 2
    return pl.pallas_call(
        _pose_body,
        grid=(n2 // _POSE_R,),
        in_specs=[
            pl.BlockSpec((_POSE_R, 32), lambda i: (i, 0)),
            pl.BlockSpec((_POSE_R, 2 * C), lambda i: (i, 0)),
            pl.BlockSpec((32, 2 * C), lambda i: (0, 0)),
            pl.BlockSpec((1, 2 * C), lambda i: (0, 0)),
            pl.BlockSpec((1, 2 * C), lambda i: (0, 0)),
            pl.BlockSpec((1, 2 * C), lambda i: (0, 0)),
            pl.BlockSpec((1, 32), lambda i: (0, 0)),
        ],
        out_specs=pl.BlockSpec((_POSE_R, 2 * C), lambda i: (i, 0)),
        out_shape=jax.ShapeDtypeStruct((n2, 2 * C), jnp.float32),
    )(pose2, partial2, W2, pb2, pw2, pbb2, scale2)


_SCALE32 = np.ones((1, 32), np.float32)
_SCALE32[0, [3, 7, 11, 19, 23, 27]] = TSCALE


def kernel(shape, color, pose, instance_id, t, pad,
           shape_table, color_table, inst_table, temp_table,
           pose_W, pose_b,
           sn_w, sn_b, cn_w, cn_b, pn_w, pn_b, in_w, in_b, tn_w, tn_b):
    del pad

    ixs = shape.reshape(TOK).astype(jnp.int32)
    ixc = color.reshape(TOK).astype(jnp.int32)
    ixn = instance_id.reshape(TOK).astype(jnp.int32)
    ixt = t.reshape(TOK).astype(jnp.int32)

    stn = _prenorm(shape_table, sn_w, sn_b, 1000)
    ctn = _prenorm(color_table, cn_w, cn_b, 1000)
    n_inst = inst_table.shape[0]
    inst_padded = jnp.concatenate(
        [inst_table, jnp.zeros((1024 - n_inst, C), jnp.float32)], axis=0)
    ntn = _prenorm(inst_padded, in_w, in_b, 512)
    ttn = _prenorm(temp_table, tn_w, tn_b, 512)

    partial = _gather_sum(stn, ctn, ntn, ttn, ixs, ixc, ixn, ixt)

    W2 = (jnp.zeros((32, 2 * C), jnp.float32)
          .at[:12, :C].set(pose_W)
          .at[16:28, C:].set(pose_W))
    pose2 = pose.reshape(TOK // 2, 32)
    partial2 = partial.reshape(TOK // 2, 2 * C)
    pb2 = jnp.concatenate([pose_b, pose_b]).reshape(1, 2 * C)
    pw2 = jnp.concatenate([pn_w, pn_w]).reshape(1, 2 * C)
    pbb2 = jnp.concatenate([pn_b, pn_b]).reshape(1, 2 * C)
    out = _pose_add(pose2, partial2, W2, pb2, pw2, pbb2,
                    jnp.asarray(_SCALE32))
    return out.reshape(S, B, C)


# speedup vs baseline: 1.9925x; 1.9925x over previous
import functools

import jax
import jax.numpy as jnp
import numpy as np
from jax import lax
from jax.experimental import pallas as pl
from jax.experimental.pallas import tpu as pltpu
from jax.experimental.pallas import tpu_sc as plsc

S, B, C = 200, 1024, 64
TOK = S * B
TSCALE = 0.005
EPS = 1e-5

NC, NS = 2, 16
NW = NC * NS
GRP = 128
GPW = TOK // (NW * GRP)
TPW = TOK // NW


def _half_ln(x, w2, b2):
    lo = x[:, :C]
    hi = x[:, C:]
    mu_lo = jnp.mean(lo, axis=-1, keepdims=True)
    mu_hi = jnp.mean(hi, axis=-1, keepdims=True)
    var_lo = jnp.mean((lo - mu_lo) ** 2, axis=-1, keepdims=True)
    var_hi = jnp.mean((hi - mu_hi) ** 2, axis=-1, keepdims=True)
    y_lo = (lo - mu_lo) * lax.rsqrt(var_lo + EPS)
    y_hi = (hi - mu_hi) * lax.rsqrt(var_hi + EPS)
    return jnp.concatenate([y_lo, y_hi], axis=-1) * w2 + b2


def _ln_rows_body(t_ref, w_ref, b_ref, o_ref):
    o_ref[...] = _half_ln(t_ref[...], w_ref[...], b_ref[...])


def _prenorm(table, w, b, block):
    n = table.shape[0]
    t2 = table.reshape(n // 2, 2 * C)
    w2 = jnp.concatenate([w, w]).reshape(1, 2 * C)
    b2 = jnp.concatenate([b, b]).reshape(1, 2 * C)
    out2 = pl.pallas_call(
        _ln_rows_body,
        grid=(n // 2 // block,),
        in_specs=[
            pl.BlockSpec((block, 2 * C), lambda i: (i, 0)),
            pl.BlockSpec((1, 2 * C), lambda i: (0, 0)),
            pl.BlockSpec((1, 2 * C), lambda i: (0, 0)),
        ],
        out_specs=pl.BlockSpec((block, 2 * C), lambda i: (i, 0)),
        out_shape=jax.ShapeDtypeStruct((n // 2, 2 * C), jnp.float32),
    )(t2, w2, b2)
    return out2.reshape(n, C)


def _make_gather_sum():
    mesh = plsc.VectorSubcoreMesh(core_axis_name="c", subcore_axis_name="s")

    @functools.partial(
        pl.kernel,
        mesh=mesh,
        compiler_params=pltpu.CompilerParams(use_tc_tiling_on_sc=False),
        out_type=jax.ShapeDtypeStruct((TOK, C), jnp.float32),
        scratch_types=[
            pltpu.VMEM((TPW,), jnp.int32),
            pltpu.VMEM((TPW,), jnp.int32),
            pltpu.VMEM((TPW,), jnp.int32),
            pltpu.VMEM((TPW,), jnp.int32),
            pltpu.VMEM((GRP, C), jnp.float32),
            pltpu.VMEM((GRP, C), jnp.float32),
            pltpu.VMEM((GRP, C), jnp.float32),
            pltpu.VMEM((GRP, C), jnp.float32),
            pltpu.SemaphoreType.DMA,
        ],
    )
    def gather_sum(st, ct, nt, tt, ixs, ixc, ixn, ixt, out,
                   vs, vc, vn, vt, rs, rc, rn, rt, sem):
        cid = lax.axis_index("c")
        sid = lax.axis_index("s")
        wid = sid * NC + cid
        tok0 = wid * TPW

        pltpu.sync_copy(ixs.at[pl.ds(tok0, TPW)], vs)
        pltpu.sync_copy(ixc.at[pl.ds(tok0, TPW)], vc)
        pltpu.sync_copy(ixn.at[pl.ds(tok0, TPW)], vn)
        pltpu.sync_copy(ixt.at[pl.ds(tok0, TPW)], vt)

        def group(g, carry):
            off = g * GRP
            tok = tok0 + off
            c1 = pltpu.async_copy(st.at[vs.at[pl.ds(off, GRP)]], rs, sem)
            c2 = pltpu.async_copy(ct.at[vc.at[pl.ds(off, GRP)]], rc, sem)
            c3 = pltpu.async_copy(nt.at[vn.at[pl.ds(off, GRP)]], rn, sem)
            c4 = pltpu.async_copy(tt.at[vt.at[pl.ds(off, GRP)]], rt, sem)
            c1.wait()
            c2.wait()
            c3.wait()
            c4.wait()

            def tok_body(j, cc):
                for jj in range(4):
                    row = j * 4 + jj
                    for q in range(C // 16):
                        sl = pl.ds(q * 16, 16)
                        v = rc[row, sl] + rn[row, sl] + rt[row, sl]
                        plsc.addupdate(rs.at[row, sl], v)
                return cc

            lax.fori_loop(0, GRP // 4, tok_body, 0)
            pltpu.sync_copy(rs, out.at[pl.ds(tok, GRP)])
            return carry

        lax.fori_loop(0, GPW, group, 0)

    return gather_sum


_gather_sum = _make_gather_sum()


def _pose_body(p_ref, part_ref, W_ref, pb_ref, w_ref, b_ref, sc_ref, o_ref):
    x = p_ref[...] * sc_ref[...]
    pe = jnp.dot(x, W_ref[...], preferred_element_type=jnp.float32) + pb_ref[...]
    mu = jnp.mean(pe, axis=-1, keepdims=True)
    var = jnp.mean((pe - mu) ** 2, axis=-1, keepdims=True)
    o_ref[...] = ((pe - mu) * lax.rsqrt(var + EPS) * w_ref[...] + b_ref[...]
                  + part_ref[...])


_POSE_R = 2048


def _pose_add(pose_flat, partial, W16, pose_b, pn_w, pn_b, scale):
    return pl.pallas_call(
        _pose_body,
        grid=(TOK // _POSE_R,),
        in_specs=[
            pl.BlockSpec((_POSE_R, 16), lambda i: (i, 0)),
            pl.BlockSpec((_POSE_R, C), lambda i: (i, 0)),
            pl.BlockSpec((16, C), lambda i: (0, 0)),
            pl.BlockSpec((1, C), lambda i: (0, 0)),
            pl.BlockSpec((1, C), lambda i: (0, 0)),
            pl.BlockSpec((1, C), lambda i: (0, 0)),
            pl.BlockSpec((1, 16), lambda i: (0, 0)),
        ],
        out_specs=pl.BlockSpec((_POSE_R, C), lambda i: (i, 0)),
        out_shape=jax.ShapeDtypeStruct((TOK, C), jnp.float32),
    )(pose_flat, partial, W16, pose_b.reshape(1, C), pn_w.reshape(1, C),
      pn_b.reshape(1, C), scale)


_SCALE16 = np.ones((1, 16), np.float32)
_SCALE16[0, [3, 7, 11]] = TSCALE


def kernel(shape, color, pose, instance_id, t, pad,
           shape_table, color_table, inst_table, temp_table,
           pose_W, pose_b,
           sn_w, sn_b, cn_w, cn_b, pn_w, pn_b, in_w, in_b, tn_w, tn_b):
    del pad

    ixs = shape.reshape(TOK).astype(jnp.int32)
    ixc = color.reshape(TOK).astype(jnp.int32)
    ixn = instance_id.reshape(TOK).astype(jnp.int32)
    ixt = t.reshape(TOK).astype(jnp.int32)

    stn = _prenorm(shape_table, sn_w, sn_b, 1000)
    ctn = _prenorm(color_table, cn_w, cn_b, 1000)
    n_inst = inst_table.shape[0]
    inst_padded = jnp.concatenate(
        [inst_table, jnp.zeros((1024 - n_inst, C), jnp.float32)], axis=0)
    ntn = _prenorm(inst_padded, in_w, in_b, 512)
    ttn = _prenorm(temp_table, tn_w, tn_b, 512)

    partial = _gather_sum(stn, ctn, ntn, ttn, ixs, ixc, ixn, ixt)

    W16 = jnp.zeros((16, C), jnp.float32).at[:12].set(pose_W)
    pose_flat = pose.reshape(TOK, 16)
    out = _pose_add(pose_flat, partial, W16, pose_b, pn_w, pn_b,
                    jnp.asarray(_SCALE16))
    return out.reshape(S, B, C)


# speedup vs baseline: 2.1210x; 1.0645x over previous
import functools

import jax
import jax.numpy as jnp
import numpy as np
from jax import lax
from jax.experimental import pallas as pl
from jax.experimental.pallas import tpu as pltpu
from jax.experimental.pallas import tpu_sc as plsc

S, B, C = 200, 1024, 64
TOK = S * B
TSCALE = 0.005
EPS = 1e-5

NC, NS = 2, 16
NW = NC * NS
GRP = 128
GPW = TOK // (NW * GRP)
TPW = TOK // NW


def _half_ln(x, w2, b2):
    lo = x[:, :C]
    hi = x[:, C:]
    mu_lo = jnp.mean(lo, axis=-1, keepdims=True)
    mu_hi = jnp.mean(hi, axis=-1, keepdims=True)
    var_lo = jnp.mean((lo - mu_lo) ** 2, axis=-1, keepdims=True)
    var_hi = jnp.mean((hi - mu_hi) ** 2, axis=-1, keepdims=True)
    y_lo = (lo - mu_lo) * lax.rsqrt(var_lo + EPS)
    y_hi = (hi - mu_hi) * lax.rsqrt(var_hi + EPS)
    return jnp.concatenate([y_lo, y_hi], axis=-1) * w2 + b2


def _ln_rows_body(t_ref, w_ref, b_ref, o_ref):
    o_ref[...] = _half_ln(t_ref[...], w_ref[...], b_ref[...])


def _prenorm(table, w, b, block):
    n = table.shape[0]
    t2 = table.reshape(n // 2, 2 * C)
    w2 = jnp.concatenate([w, w]).reshape(1, 2 * C)
    b2 = jnp.concatenate([b, b]).reshape(1, 2 * C)
    out2 = pl.pallas_call(
        _ln_rows_body,
        grid=(n // 2 // block,),
        in_specs=[
            pl.BlockSpec((block, 2 * C), lambda i: (i, 0)),
            pl.BlockSpec((1, 2 * C), lambda i: (0, 0)),
            pl.BlockSpec((1, 2 * C), lambda i: (0, 0)),
        ],
        out_specs=pl.BlockSpec((block, 2 * C), lambda i: (i, 0)),
        out_shape=jax.ShapeDtypeStruct((n // 2, 2 * C), jnp.float32),
    )(t2, w2, b2)
    return out2.reshape(n, C)


def _make_gather_sum():
    mesh = plsc.VectorSubcoreMesh(core_axis_name="c", subcore_axis_name="s")

    @functools.partial(
        pl.kernel,
        mesh=mesh,
        compiler_params=pltpu.CompilerParams(use_tc_tiling_on_sc=False),
        out_type=jax.ShapeDtypeStruct((TOK, C), jnp.float32),
        scratch_types=[
            pltpu.VMEM((TPW,), jnp.int32),
            pltpu.VMEM((TPW,), jnp.int32),
            pltpu.VMEM((TPW,), jnp.int32),
            pltpu.VMEM((TPW,), jnp.int32),
            pltpu.VMEM((2, GRP, C), jnp.float32),
            pltpu.VMEM((2, GRP, C), jnp.float32),
            pltpu.VMEM((2, GRP, C), jnp.float32),
            pltpu.VMEM((2, GRP, C), jnp.float32),
            pltpu.SemaphoreType.DMA,
            pltpu.SemaphoreType.DMA,
        ],
    )
    def gather_sum(st, ct, nt, tt, ixs, ixc, ixn, ixt, out,
                   vs, vc, vn, vt, rs, rc, rn, rt, semA, semB):
        cid = lax.axis_index("c")
        sid = lax.axis_index("s")
        wid = sid * NC + cid
        tok0 = wid * TPW

        pltpu.sync_copy(ixs.at[pl.ds(tok0, TPW)], vs)
        pltpu.sync_copy(ixc.at[pl.ds(tok0, TPW)], vc)
        pltpu.sync_copy(ixn.at[pl.ds(tok0, TPW)], vn)
        pltpu.sync_copy(ixt.at[pl.ds(tok0, TPW)], vt)

        sems = (semA, semB)

        def issue(g, b):
            off = g * GRP
            pltpu.async_copy(st.at[vs.at[pl.ds(off, GRP)]], rs.at[b], sems[b])
            pltpu.async_copy(ct.at[vc.at[pl.ds(off, GRP)]], rc.at[b], sems[b])
            pltpu.async_copy(nt.at[vn.at[pl.ds(off, GRP)]], rn.at[b], sems[b])
            pltpu.async_copy(tt.at[vt.at[pl.ds(off, GRP)]], rt.at[b], sems[b])

        def drain(b):
            for dst in (rs, rc, rn, rt):
                pltpu.make_async_copy(st.at[pl.ds(0, GRP)], dst.at[b],
                                      sems[b]).wait()

        def consume(g, b):
            drain(b)

            def tok_body(j, cc):
                for jj in range(4):
                    row = j * 4 + jj
                    for q in range(C // 16):
                        sl = pl.ds(q * 16, 16)
                        v = (rc[b, row, sl] + rn[b, row, sl]
                             + rt[b, row, sl])
                        plsc.addupdate(rs.at[b, row, sl], v)
                return cc

            lax.fori_loop(0, GRP // 4, tok_body, 0)
            pltpu.sync_copy(rs.at[b], out.at[pl.ds(tok0 + g * GRP, GRP)])

        issue(0, 0)

        def pair(h, carry):
            g0 = 2 * h
            issue(g0 + 1, 1)
            consume(g0, 0)

            @pl.when(h < GPW // 2 - 1)
            def _():
                issue(g0 + 2, 0)

            consume(g0 + 1, 1)
            return carry

        lax.fori_loop(0, GPW // 2, pair, 0)

    return gather_sum


_gather_sum = _make_gather_sum()


def _pose_body(p_ref, part_ref, W_ref, pb_ref, w_ref, b_ref, sc_ref, o_ref):
    x = p_ref[...] * sc_ref[...]
    pe = jnp.dot(x, W_ref[...], preferred_element_type=jnp.float32) + pb_ref[...]
    mu = jnp.mean(pe, axis=-1, keepdims=True)
    var = jnp.mean((pe - mu) ** 2, axis=-1, keepdims=True)
    o_ref[...] = ((pe - mu) * lax.rsqrt(var + EPS) * w_ref[...] + b_ref[...]
                  + part_ref[...])


_POSE_R = 2048


def _pose_add(pose_flat, partial, W16, pose_b, pn_w, pn_b, scale):
    return pl.pallas_call(
        _pose_body,
        grid=(TOK // _POSE_R,),
        in_specs=[
            pl.BlockSpec((_POSE_R, 16), lambda i: (i, 0)),
            pl.BlockSpec((_POSE_R, C), lambda i: (i, 0)),
            pl.BlockSpec((16, C), lambda i: (0, 0)),
            pl.BlockSpec((1, C), lambda i: (0, 0)),
            pl.BlockSpec((1, C), lambda i: (0, 0)),
            pl.BlockSpec((1, C), lambda i: (0, 0)),
            pl.BlockSpec((1, 16), lambda i: (0, 0)),
        ],
        out_specs=pl.BlockSpec((_POSE_R, C), lambda i: (i, 0)),
        out_shape=jax.ShapeDtypeStruct((TOK, C), jnp.float32),
    )(pose_flat, partial, W16, pose_b.reshape(1, C), pn_w.reshape(1, C),
      pn_b.reshape(1, C), scale)


_SCALE16 = np.ones((1, 16), np.float32)
_SCALE16[0, [3, 7, 11]] = TSCALE


def kernel(shape, color, pose, instance_id, t, pad,
           shape_table, color_table, inst_table, temp_table,
           pose_W, pose_b,
           sn_w, sn_b, cn_w, cn_b, pn_w, pn_b, in_w, in_b, tn_w, tn_b):
    del pad

    ixs = shape.reshape(TOK).astype(jnp.int32)
    ixc = color.reshape(TOK).astype(jnp.int32)
    ixn = instance_id.reshape(TOK).astype(jnp.int32)
    ixt = t.reshape(TOK).astype(jnp.int32)

    stn = _prenorm(shape_table, sn_w, sn_b, 1000)
    ctn = _prenorm(color_table, cn_w, cn_b, 1000)
    n_inst = inst_table.shape[0]
    inst_padded = jnp.concatenate(
        [inst_table, jnp.zeros((1024 - n_inst, C), jnp.float32)], axis=0)
    ntn = _prenorm(inst_padded, in_w, in_b, 512)
    ttn = _prenorm(temp_table, tn_w, tn_b, 512)

    partial = _gather_sum(stn, ctn, ntn, ttn, ixs, ixc, ixn, ixt)

    W16 = jnp.zeros((16, C), jnp.float32).at[:12].set(pose_W)
    pose_flat = pose.reshape(TOK, 16)
    out = _pose_add(pose_flat, partial, W16, pose_b, pn_w, pn_b,
                    jnp.asarray(_SCALE16))
    return out.reshape(S, B, C)


# speedup vs baseline: 2.3682x; 1.1165x over previous
import functools

import jax
import jax.numpy as jnp
import numpy as np
from jax import lax
from jax.experimental import pallas as pl
from jax.experimental.pallas import tpu as pltpu
from jax.experimental.pallas import tpu_sc as plsc

S, B, C = 200, 1024, 64
TOK = S * B
TSCALE = 0.005
EPS = 1e-5

NC, NS = 2, 16
NW = NC * NS
GRP = 128
GPW = TOK // (NW * GRP)
TPW = TOK // NW


_MMEAN = np.zeros((128, 128), np.float32)
_MMEAN[:64, :64] = 1.0 / 64.0
_MMEAN[64:, 64:] = 1.0 / 64.0


def _half_ln(x, w2, b2, mm):
    mu = jnp.dot(x, mm, preferred_element_type=jnp.float32)
    c = x - mu
    var = jnp.dot(c * c, mm, preferred_element_type=jnp.float32)
    return c * lax.rsqrt(var + EPS) * w2 + b2


def _ln_rows_body(t_ref, w_ref, b_ref, mm_ref, o_ref):
    o_ref[...] = _half_ln(t_ref[...], w_ref[...], b_ref[...], mm_ref[...])


def _prenorm(table, w, b, block):
    n = table.shape[0]
    t2 = table.reshape(n // 2, 2 * C)
    w2 = jnp.concatenate([w, w]).reshape(1, 2 * C)
    b2 = jnp.concatenate([b, b]).reshape(1, 2 * C)
    out2 = pl.pallas_call(
        _ln_rows_body,
        grid=(n // 2 // block,),
        in_specs=[
            pl.BlockSpec((block, 2 * C), lambda i: (i, 0)),
            pl.BlockSpec((1, 2 * C), lambda i: (0, 0)),
            pl.BlockSpec((1, 2 * C), lambda i: (0, 0)),
            pl.BlockSpec((2 * C, 2 * C), lambda i: (0, 0)),
        ],
        out_specs=pl.BlockSpec((block, 2 * C), lambda i: (i, 0)),
        out_shape=jax.ShapeDtypeStruct((n // 2, 2 * C), jnp.float32),
    )(t2, w2, b2, jnp.asarray(_MMEAN))
    return out2.reshape(n, C)


def _make_gather_sum():
    mesh = plsc.VectorSubcoreMesh(core_axis_name="c", subcore_axis_name="s")

    @functools.partial(
        pl.kernel,
        mesh=mesh,
        compiler_params=pltpu.CompilerParams(use_tc_tiling_on_sc=False),
        out_type=jax.ShapeDtypeStruct((TOK, C), jnp.float32),
        scratch_types=[
            pltpu.VMEM((TPW,), jnp.int32),
            pltpu.VMEM((TPW,), jnp.int32),
            pltpu.VMEM((TPW,), jnp.int32),
            pltpu.VMEM((TPW,), jnp.int32),
            pltpu.VMEM((2, GRP, C), jnp.float32),
            pltpu.VMEM((2, GRP, C), jnp.float32),
            pltpu.VMEM((2, GRP, C), jnp.float32),
            pltpu.VMEM((2, GRP, C), jnp.float32),
            pltpu.SemaphoreType.DMA,
            pltpu.SemaphoreType.DMA,
        ],
    )
    def gather_sum(st, ct, nt, tt, ixs, ixc, ixn, ixt, out,
                   vs, vc, vn, vt, rs, rc, rn, rt, semA, semB):
        cid = lax.axis_index("c")
        sid = lax.axis_index("s")
        wid = sid * NC + cid
        tok0 = wid * TPW

        pltpu.sync_copy(ixs.at[pl.ds(tok0, TPW)], vs)
        pltpu.sync_copy(ixc.at[pl.ds(tok0, TPW)], vc)
        pltpu.sync_copy(ixn.at[pl.ds(tok0, TPW)], vn)
        pltpu.sync_copy(ixt.at[pl.ds(tok0, TPW)], vt)

        sems = (semA, semB)

        def issue(g, b):
            off = g * GRP
            pltpu.async_copy(st.at[vs.at[pl.ds(off, GRP)]], rs.at[b], sems[b])
            pltpu.async_copy(ct.at[vc.at[pl.ds(off, GRP)]], rc.at[b], sems[b])
            pltpu.async_copy(nt.at[vn.at[pl.ds(off, GRP)]], rn.at[b], sems[b])
            pltpu.async_copy(tt.at[vt.at[pl.ds(off, GRP)]], rt.at[b], sems[b])

        def drain(b):
            for dst in (rs, rc, rn, rt):
                pltpu.make_async_copy(st.at[pl.ds(0, GRP)], dst.at[b],
                                      sems[b]).wait()

        def consume(g, b):
            drain(b)

            def tok_body(j, cc):
                for jj in range(8):
                    row = j * 8 + jj
                    for q in range(C // 16):
                        sl = pl.ds(q * 16, 16)
                        v = (rc[b, row, sl] + rn[b, row, sl]
                             + rt[b, row, sl])
                        plsc.addupdate(rs.at[b, row, sl], v)
                return cc

            lax.fori_loop(0, GRP // 8, tok_body, 0)
            pltpu.sync_copy(rs.at[b], out.at[pl.ds(tok0 + g * GRP, GRP)])

        issue(0, 0)

        def pair(h, carry):
            g0 = 2 * h
            issue(g0 + 1, 1)
            consume(g0, 0)

            @pl.when(h < GPW // 2 - 1)
            def _():
                issue(g0 + 2, 0)

            consume(g0 + 1, 1)
            return carry

        lax.fori_loop(0, GPW // 2, pair, 0)

    return gather_sum


_gather_sum = _make_gather_sum()


def _pose_body(p_ref, part_ref, W_ref, pb_ref, w_ref, b_ref, sc_ref, mm_ref,
               o_ref):
    x = p_ref[...] * sc_ref[...]
    pe = jnp.dot(x, W_ref[...], preferred_element_type=jnp.float32) + pb_ref[...]
    o_ref[...] = _half_ln(pe, w_ref[...], b_ref[...], mm_ref[...]) + part_ref[...]


_POSE_R = 2048


def _pose_add(pose2, partial2, W2, pb2, pw2, pbb2, scale2):
    n2 = TOK // 2
    return pl.pallas_call(
        _pose_body,
        grid=(n2 // _POSE_R,),
        in_specs=[
            pl.BlockSpec((_POSE_R, 32), lambda i: (i, 0)),
            pl.BlockSpec((_POSE_R, 2 * C), lambda i: (i, 0)),
            pl.BlockSpec((32, 2 * C), lambda i: (0, 0)),
            pl.BlockSpec((1, 2 * C), lambda i: (0, 0)),
            pl.BlockSpec((1, 2 * C), lambda i: (0, 0)),
            pl.BlockSpec((1, 2 * C), lambda i: (0, 0)),
            pl.BlockSpec((1, 32), lambda i: (0, 0)),
            pl.BlockSpec((2 * C, 2 * C), lambda i: (0, 0)),
        ],
        out_specs=pl.BlockSpec((_POSE_R, 2 * C), lambda i: (i, 0)),
        out_shape=jax.ShapeDtypeStruct((n2, 2 * C), jnp.float32),
    )(pose2, partial2, W2, pb2, pw2, pbb2, scale2, jnp.asarray(_MMEAN))


_SCALE32 = np.ones((1, 32), np.float32)
_SCALE32[0, [3, 7, 11, 19, 23, 27]] = TSCALE


def kernel(shape, color, pose, instance_id, t, pad,
           shape_table, color_table, inst_table, temp_table,
           pose_W, pose_b,
           sn_w, sn_b, cn_w, cn_b, pn_w, pn_b, in_w, in_b, tn_w, tn_b):
    del pad

    ixs = shape.reshape(TOK).astype(jnp.int32)
    ixc = color.reshape(TOK).astype(jnp.int32)
    ixn = instance_id.reshape(TOK).astype(jnp.int32)
    ixt = t.reshape(TOK).astype(jnp.int32)

    stn = _prenorm(shape_table, sn_w, sn_b, 1000)
    ctn = _prenorm(color_table, cn_w, cn_b, 1000)
    n_inst = inst_table.shape[0]
    inst_padded = jnp.concatenate(
        [inst_table, jnp.zeros((1024 - n_inst, C), jnp.float32)], axis=0)
    ntn = _prenorm(inst_padded, in_w, in_b, 512)
    ttn = _prenorm(temp_table, tn_w, tn_b, 512)

    partial = _gather_sum(stn, ctn, ntn, ttn, ixs, ixc, ixn, ixt)

    W2 = (jnp.zeros((32, 2 * C), jnp.float32)
          .at[:12, :C].set(pose_W)
          .at[16:28, C:].set(pose_W))
    pose16 = pose.reshape(TOK, 16)
    pose2 = lax.optimization_barrier(pose16).reshape(TOK // 2, 32)
    partial2 = partial.reshape(TOK // 2, 2 * C)
    pb2 = jnp.concatenate([pose_b, pose_b]).reshape(1, 2 * C)
    pw2 = jnp.concatenate([pn_w, pn_w]).reshape(1, 2 * C)
    pbb2 = jnp.concatenate([pn_b, pn_b]).reshape(1, 2 * C)
    out = _pose_add(pose2, partial2, W2, pb2, pw2, pbb2,
                    jnp.asarray(_SCALE32))
    return out.reshape(S, B, C)
